# baseline (speedup 1.0000x reference)
import functools

import jax
import jax.numpy as jnp
from jax import lax
from jax.experimental import pallas as pl
from jax.experimental.pallas import tpu as pltpu
from jax.experimental.pallas import tpu_sc as plsc

WD = 0.0001
D = 64
CHUNK = 128


def _sc_gather_dot(u, i, j, W, H):
    B = u.shape[0]
    info = plsc.get_sparse_core_info()
    NC, NS, L = info.num_cores, info.num_subcores, info.num_lanes
    NW = NC * NS
    BPW = B // NW
    NCHUNK = BPW // CHUNK
    NGROUP = BPW // L

    mesh = plsc.VectorSubcoreMesh(core_axis_name="c", subcore_axis_name="s")

    @functools.partial(
        pl.kernel,
        out_type=[
            jax.ShapeDtypeStruct((B,), jnp.float32),
            jax.ShapeDtypeStruct((NW * L,), jnp.float32),
        ],
        mesh=mesh,
        compiler_params=pltpu.CompilerParams(needs_layout_passes=False,
                                             use_tc_tiling_on_sc=False),
        scratch_types=[
            pltpu.VMEM((NCHUNK, CHUNK), jnp.int32),
            pltpu.VMEM((NCHUNK, CHUNK), jnp.int32),
            pltpu.VMEM((NCHUNK, CHUNK), jnp.int32),
            pltpu.VMEM((BPW, D), jnp.float32),
            pltpu.VMEM((BPW, D), jnp.float32),
            pltpu.VMEM((BPW, D), jnp.float32),
            pltpu.VMEM((BPW,), jnp.float32),
            pltpu.VMEM((L,), jnp.float32),
            pltpu.SemaphoreType.DMA,
            pltpu.SemaphoreType.DMA,
            pltpu.SemaphoreType.DMA,
        ],
    )
    def sc_kernel(u_hbm, i_hbm, j_hbm, W_hbm, H_hbm, x_hbm, sq_hbm,
                  u_idx, i_idx, j_idx, u_rows, i_rows, j_rows, x_v, sq_v,
                  su, si, sj):
        wid = lax.axis_index("s") * NC + lax.axis_index("c")
        base = wid * BPW

        for k in range(NCHUNK):
            pltpu.sync_copy(u_hbm.at[pl.ds(base + k * CHUNK, CHUNK)], u_idx.at[k])
            pltpu.sync_copy(i_hbm.at[pl.ds(base + k * CHUNK, CHUNK)], i_idx.at[k])
            pltpu.sync_copy(j_hbm.at[pl.ds(base + k * CHUNK, CHUNK)], j_idx.at[k])
        copies = []
        for k in range(NCHUNK):
            dst = pl.ds(k * CHUNK, CHUNK)
            copies.append(pltpu.async_copy(W_hbm.at[u_idx.at[k]], u_rows.at[dst], su))
            copies.append(pltpu.async_copy(H_hbm.at[i_idx.at[k]], i_rows.at[dst], si))
            copies.append(pltpu.async_copy(H_hbm.at[j_idx.at[k]], j_rows.at[dst], sj))
        for c in copies:
            c.wait()

        lanes = lax.iota(jnp.int32, L)

        def group_body(g, sq_acc):
            svec = jnp.zeros((L,), jnp.float32)
            for r in range(L):
                row = g * L + r
                acc = jnp.zeros((L,), jnp.float32)
                for c in range(D // L):
                    sl = pl.ds(c * L, L)
                    uv = u_rows[row, sl]
                    iv = i_rows[row, sl]
                    jv = j_rows[row, sl]
                    acc = acc + uv * (iv - jv)
                    sq_acc = sq_acc + (uv * uv + (iv * iv + jv * jv))
                s = jnp.sum(acc)
                svec = jnp.where(lanes == r, s, svec)
            x_v[pl.ds(g * L, L)] = svec
            return sq_acc

        sq_acc = lax.fori_loop(0, NGROUP, group_body,
                               jnp.zeros((L,), jnp.float32))
        sq_v[...] = sq_acc
        pltpu.sync_copy(x_v, x_hbm.at[pl.ds(base, BPW)])
        pltpu.sync_copy(sq_v, sq_hbm.at[pl.ds(wid * L, L)])

    return sc_kernel(u, i, j, W, H)


def _tc_finish(x2d, sq2d):
    def body(x_ref, sq_ref, o_ref):
        x = x_ref[...]
        ls = jnp.minimum(x, 0.0) - jnp.log1p(jnp.exp(-jnp.abs(x)))
        o_ref[0, 0] = WD * jnp.sum(sq_ref[...]) - jnp.sum(ls)

    return pl.pallas_call(
        body,
        out_shape=jax.ShapeDtypeStruct((1, 1), jnp.float32),
        out_specs=pl.BlockSpec(memory_space=pltpu.SMEM),
    )(x2d, sq2d)


def kernel(u, i, j, W, H):
    u = u.astype(jnp.int32)
    i = i.astype(jnp.int32)
    j = j.astype(jnp.int32)
    x, sq = _sc_gather_dot(u, i, j, W, H)
    out = _tc_finish(x.reshape(128, -1), sq.reshape(4, -1))
    return out[0, 0]


# speedup vs baseline: 1.6223x; 1.6223x over previous
import functools

import jax
import jax.numpy as jnp
from jax import lax
from jax.experimental import pallas as pl
from jax.experimental.pallas import tpu as pltpu
from jax.experimental.pallas import tpu_sc as plsc

WD = 0.0001
D = 64
ICH = 128


def _sc_gather_dot(u, i, j, W, H):
    B = u.shape[0]
    info = plsc.get_sparse_core_info()
    NC, NS, L = info.num_cores, info.num_subcores, info.num_lanes
    NW = NC * NS
    BPW = B // NW
    NCHUNK = BPW // ICH
    NGROUP = BPW // L

    mesh = plsc.VectorSubcoreMesh(core_axis_name="c", subcore_axis_name="s")

    @functools.partial(
        pl.kernel,
        out_type=[
            jax.ShapeDtypeStruct((B,), jnp.float32),
            jax.ShapeDtypeStruct((NW * L,), jnp.float32),
        ],
        mesh=mesh,
        compiler_params=pltpu.CompilerParams(needs_layout_passes=False),
        scratch_types=[
            pltpu.VMEM((ICH,), jnp.int32),
            pltpu.VMEM((ICH,), jnp.int32),
            pltpu.VMEM((ICH,), jnp.int32),
            pltpu.VMEM((ICH, D), jnp.float32),
            pltpu.VMEM((ICH, D), jnp.float32),
            pltpu.VMEM((ICH, D), jnp.float32),
            pltpu.VMEM((BPW,), jnp.float32),
            pltpu.VMEM((L,), jnp.float32),
            pltpu.SemaphoreType.DMA,
            pltpu.SemaphoreType.DMA,
            pltpu.SemaphoreType.DMA,
        ],
    )
    def sc_kernel(u_hbm, i_hbm, j_hbm, W_hbm, H_hbm, x_hbm, sq_hbm,
                  u_idx_v, i_idx_v, j_idx_v,
                  u_rows, i_rows, j_rows, x_v, sq_v,
                  su, si, sj):
        wid = lax.axis_index("s") * NC + lax.axis_index("c")
        base = wid * BPW

        lanes = lax.iota(jnp.int32, L)

        def chunk_body(k, sq_acc):
            off = base + k * ICH
            pltpu.sync_copy(u_hbm.at[pl.ds(off, ICH)], u_idx_v)
            pltpu.sync_copy(i_hbm.at[pl.ds(off, ICH)], i_idx_v)
            pltpu.sync_copy(j_hbm.at[pl.ds(off, ICH)], j_idx_v)

            def row16_body(r16, _):
                rowbase = r16 * L
                uvec = u_idx_v[pl.ds(rowbase, L)]
                ivec = i_idx_v[pl.ds(rowbase, L)]
                jvec = j_idx_v[pl.ds(rowbase, L)]
                zero = jnp.zeros((L,), jnp.int32)
                for r in range(L):
                    us = jnp.sum(jnp.where(lanes == r, uvec, zero))
                    is_ = jnp.sum(jnp.where(lanes == r, ivec, zero))
                    js = jnp.sum(jnp.where(lanes == r, jvec, zero))
                    dst = pl.ds(rowbase + r, 1)
                    pltpu.async_copy(W_hbm.at[pl.ds(us, 1)],
                                     u_rows.at[dst], su)
                    pltpu.async_copy(H_hbm.at[pl.ds(is_, 1)],
                                     i_rows.at[dst], si)
                    pltpu.async_copy(H_hbm.at[pl.ds(js, 1)],
                                     j_rows.at[dst], sj)
                return 0

            lax.fori_loop(0, ICH // L, row16_body, 0)
            pltpu.make_async_copy(W_hbm.at[pl.ds(0, ICH)], u_rows, su).wait()
            pltpu.make_async_copy(H_hbm.at[pl.ds(0, ICH)], i_rows, si).wait()
            pltpu.make_async_copy(H_hbm.at[pl.ds(0, ICH)], j_rows, sj).wait()

            def group_body(g, sq_acc):
                svec = jnp.zeros((L,), jnp.float32)
                for r in range(L):
                    row = g * L + r
                    acc = jnp.zeros((L,), jnp.float32)
                    for c in range(D // L):
                        sl = pl.ds(c * L, L)
                        uv = u_rows[row, sl]
                        iv = i_rows[row, sl]
                        jv = j_rows[row, sl]
                        acc = acc + uv * (iv - jv)
                        sq_acc = sq_acc + (uv * uv + (iv * iv + jv * jv))
                    s = jnp.sum(acc)
                    svec = jnp.where(lanes == r, s, svec)
                x_v[pl.ds(k * ICH + g * L, L)] = svec
                return sq_acc

            return lax.fori_loop(0, ICH // L, group_body, sq_acc)

        sq_acc = lax.fori_loop(0, NCHUNK, chunk_body,
                               jnp.zeros((L,), jnp.float32))
        sq_v[...] = sq_acc
        pltpu.sync_copy(x_v, x_hbm.at[pl.ds(base, BPW)])
        pltpu.sync_copy(sq_v, sq_hbm.at[pl.ds(wid * L, L)])

    return sc_kernel(u, i, j, W, H)


def _tc_finish(x2d, sq2d):
    def body(x_ref, sq_ref, o_ref):
        x = x_ref[...]
        ls = jnp.minimum(x, 0.0) - jnp.log1p(jnp.exp(-jnp.abs(x)))
        o_ref[0, 0] = WD * jnp.sum(sq_ref[...]) - jnp.sum(ls)

    return pl.pallas_call(
        body,
        out_shape=jax.ShapeDtypeStruct((1, 1), jnp.float32),
        out_specs=pl.BlockSpec(memory_space=pltpu.SMEM),
    )(x2d, sq2d)


def kernel(u, i, j, W, H):
    u = u.astype(jnp.int32)
    i = i.astype(jnp.int32)
    j = j.astype(jnp.int32)
    x, sq = _sc_gather_dot(u, i, j, W, H)
    out = _tc_finish(x.reshape(128, -1), sq.reshape(4, -1))
    return out[0, 0]
